# baseline (speedup 1.0000x reference)
import functools

import jax
import jax.numpy as jnp
from jax import lax
from jax.experimental import pallas as pl
from jax.experimental.pallas import tpu as pltpu
from jax.experimental.pallas import tpu_sc as plsc

D_MODEL = 64
LANES = 16
CHUNK = 128
EPS_OVER_D = 1e-5 / 64.0
NBUF = 4
UNROLL = 8
NEWTON_ITERS = 2


def _body(x_hbm, table_hbm, gamma_hbm, beta_hbm, out_hbm,
          idx_v, rv0, rv1, rv2, rv3, ov0, ov1, ov2, ov3, gb_v,
          sg0, sg1, sg2, sg3, ss0, ss1, ss2, ss3):
    nc = 2
    wid = lax.axis_index("s") * nc + lax.axis_index("c")
    n_chunks = idx_v.shape[0]
    base = wid * n_chunks
    half = CHUNK // 2
    rvs = [rv0, rv1, rv2, rv3]
    ovs = [ov0, ov1, ov2, ov3]
    sgs = [sg0, sg1, sg2, sg3]
    sss = [ss0, ss1, ss2, ss3]

    pltpu.sync_copy(x_hbm.at[pl.ds(base, n_chunks)], idx_v)
    pltpu.sync_copy(gamma_hbm, gb_v.at[0])
    pltpu.sync_copy(beta_hbm, gb_v.at[1])

    nq = D_MODEL // LANES
    last_lane = jnp.full((LANES,), LANES - 1, jnp.int32)
    gvecs = [gb_v[0, pl.ds(k * LANES, LANES)] for k in range(nq)]
    bvecs = [gb_v[1, pl.ds(k * LANES, LANES)] for k in range(nq)]

    def g_copy(j, p):
        return pltpu.make_async_copy(table_hbm.at[idx_v.at[j]], rvs[p], sgs[p])

    def s_copy(j, p):
        return pltpu.make_async_copy(
            ovs[p], out_hbm.at[pl.ds((base + j) * half, half)], sss[p])

    def compute(rows_v, out_v):
        def row_block(i, c):
            for u in range(UNROLL):
                r = i * UNROLL + u
                m = i * (UNROLL // 2) + u // 2
                off = (u % 2) * D_MODEL
                qs = [rows_v[r, pl.ds(k * LANES, LANES)] for k in range(nq)]
                t = (qs[0] + qs[1]) + (qs[2] + qs[3])
                t2 = (qs[0] * qs[0] + qs[1] * qs[1]) + (
                    qs[2] * qs[2] + qs[3] * qs[3])
                s = jnp.take_along_axis(jnp.cumsum(t), last_lane, axis=0)
                s2 = jnp.take_along_axis(jnp.cumsum(t2), last_lane, axis=0)
                mean = s * (1.0 / D_MODEL)
                var = s2 * (1.0 / D_MODEL) - mean * mean
                tv = var + EPS_OVER_D
                ti = plsc.bitcast(tv, jnp.int32)
                yi = 0x5F3759DF - lax.shift_right_logical(ti, 1)
                y = plsc.bitcast(yi, jnp.float32)
                half_t = tv * 0.5
                for _ in range(NEWTON_ITERS):
                    y = y * (1.5 - half_t * y * y)
                cshift = mean * y
                for k in range(nq):
                    o = qs[k] * y - cshift
                    out_v[m, pl.ds(off + k * LANES, LANES)] = (
                        o * gvecs[k] + bvecs[k])
            return c

        lax.fori_loop(0, CHUNK // UNROLL, row_block, 0)

    g_copy(0, 0).start()
    g_copy(1, 1).start()

    n_outer = n_chunks // NBUF

    def outer(jj, carry):
        for p in range(NBUF):
            j = jj * NBUF + p
            q = (p + 2) % NBUF
            g_copy(j, p).wait()
            compute(rvs[p], ovs[p])
            s_copy(j, p).start()
            if p < 2:
                @pl.when(jj >= 1)
                def _wait_prev():
                    s_copy(j - 2, q).wait()
                g_copy(j + 2, q).start()
            else:
                s_copy(j - 2, q).wait()

                @pl.when(jj <= n_outer - 2)
                def _start_next():
                    g_copy(j + 2, q).start()
        return carry

    lax.fori_loop(0, n_outer, outer, 0)
    s_copy(n_chunks - 2, 2).wait()
    s_copy(n_chunks - 1, 3).wait()


def kernel(x, table, gamma, beta):
    b, s = x.shape
    n_tok = b * s
    n_workers = 32
    per_worker = n_tok // n_workers
    n_chunks = per_worker // CHUNK
    x2d = x.reshape(n_tok // CHUNK, CHUNK).astype(jnp.int32)

    mesh = plsc.VectorSubcoreMesh(core_axis_name="c", subcore_axis_name="s")
    kern = functools.partial(
        pl.kernel,
        mesh=mesh,
        compiler_params=pltpu.CompilerParams(
            use_tc_tiling_on_sc=False, needs_layout_passes=False),
        out_type=jax.ShapeDtypeStruct((n_tok // 2, 2 * D_MODEL), jnp.float32),
        scratch_types=(
            [pltpu.VMEM((n_chunks, CHUNK), jnp.int32)]
            + [pltpu.VMEM((CHUNK, D_MODEL), jnp.float32) for _ in range(NBUF)]
            + [pltpu.VMEM((CHUNK // 2, 2 * D_MODEL), jnp.float32)
               for _ in range(NBUF)]
            + [pltpu.VMEM((2, D_MODEL), jnp.float32)]
            + [pltpu.SemaphoreType.DMA for _ in range(2 * NBUF)]
        ),
    )(_body)
    out = kern(x2d, table, gamma, beta)
    return out.reshape(b, s, D_MODEL)


# speedup vs baseline: 1.0497x; 1.0497x over previous
import functools

import jax
import jax.numpy as jnp
from jax import lax
from jax.experimental import pallas as pl
from jax.experimental.pallas import tpu as pltpu
from jax.experimental.pallas import tpu_sc as plsc

D_MODEL = 64
LANES = 16
CHUNK = 128
EPS_OVER_D = 1e-5 / 64.0
NBUF = 8
LOOKAHEAD = 4
UNROLL = 8
NEWTON_ITERS = 2


def _body(x_hbm, table_hbm, gamma_hbm, beta_hbm, out_hbm, idx_v, *scratch):
    nc = 2
    wid = lax.axis_index("s") * nc + lax.axis_index("c")
    n_chunks = idx_v.shape[0]
    base = wid * n_chunks
    rvs = list(scratch[:NBUF])
    gb_v = scratch[NBUF]
    sgs = list(scratch[NBUF + 1:2 * NBUF + 1])
    sss = list(scratch[2 * NBUF + 1:3 * NBUF + 1])

    pltpu.sync_copy(x_hbm.at[pl.ds(base, n_chunks)], idx_v)
    pltpu.sync_copy(gamma_hbm, gb_v.at[0])
    pltpu.sync_copy(beta_hbm, gb_v.at[1])

    nq = D_MODEL // LANES
    last_lane = jnp.full((LANES,), LANES - 1, jnp.int32)
    gvecs = [gb_v[0, pl.ds(k * LANES, LANES)] for k in range(nq)]
    bvecs = [gb_v[1, pl.ds(k * LANES, LANES)] for k in range(nq)]

    def g_copy(j, p):
        return pltpu.make_async_copy(table_hbm.at[idx_v.at[j]], rvs[p], sgs[p])

    def s_copy(j, p):
        return pltpu.make_async_copy(
            rvs[p], out_hbm.at[pl.ds((base + j) * CHUNK, CHUNK)], sss[p])

    def compute(rows_v):
        def row_block(i, c):
            for u in range(UNROLL):
                r = i * UNROLL + u
                qs = [rows_v[r, pl.ds(k * LANES, LANES)] for k in range(nq)]
                t = (qs[0] + qs[1]) + (qs[2] + qs[3])
                t2 = (qs[0] * qs[0] + qs[1] * qs[1]) + (
                    qs[2] * qs[2] + qs[3] * qs[3])
                s = jnp.take_along_axis(jnp.cumsum(t), last_lane, axis=0)
                s2 = jnp.take_along_axis(jnp.cumsum(t2), last_lane, axis=0)
                mean = s * (1.0 / D_MODEL)
                var = s2 * (1.0 / D_MODEL) - mean * mean
                tv = var + EPS_OVER_D
                ti = plsc.bitcast(tv, jnp.int32)
                yi = 0x5F3759DF - lax.shift_right_logical(ti, 1)
                y = plsc.bitcast(yi, jnp.float32)
                half_t = tv * 0.5
                for _ in range(NEWTON_ITERS):
                    y = y * (1.5 - half_t * y * y)
                cshift = mean * y
                for k in range(nq):
                    o = qs[k] * y - cshift
                    rows_v[r, pl.ds(k * LANES, LANES)] = o * gvecs[k] + bvecs[k]
            return c

        lax.fori_loop(0, CHUNK // UNROLL, row_block, 0)

    for p in range(LOOKAHEAD):
        g_copy(p, p).start()

    n_outer = n_chunks // NBUF

    def outer(jj, carry):
        for p in range(NBUF):
            j = jj * NBUF + p
            q = (p + LOOKAHEAD) % NBUF
            g_copy(j, p).wait()
            compute(rvs[p])
            s_copy(j, p).start()
            if p < LOOKAHEAD:
                @pl.when(jj >= 1)
                def _wait_prev():
                    s_copy(j - LOOKAHEAD, q).wait()
                g_copy(j + LOOKAHEAD, q).start()
            else:
                s_copy(j - LOOKAHEAD, q).wait()

                @pl.when(jj <= n_outer - 2)
                def _start_next():
                    g_copy(j + LOOKAHEAD, q).start()
        return carry

    lax.fori_loop(0, n_outer, outer, 0)
    for p in range(LOOKAHEAD):
        s_copy(n_chunks - LOOKAHEAD + p, NBUF - LOOKAHEAD + p).wait()


def kernel(x, table, gamma, beta):
    b, s = x.shape
    n_tok = b * s
    n_workers = 32
    per_worker = n_tok // n_workers
    n_chunks = per_worker // CHUNK
    x2d = x.reshape(n_tok // CHUNK, CHUNK).astype(jnp.int32)

    mesh = plsc.VectorSubcoreMesh(core_axis_name="c", subcore_axis_name="s")
    kern = functools.partial(
        pl.kernel,
        mesh=mesh,
        compiler_params=pltpu.CompilerParams(
            use_tc_tiling_on_sc=False, needs_layout_passes=False),
        out_type=jax.ShapeDtypeStruct((n_tok, D_MODEL), jnp.float32),
        scratch_types=(
            [pltpu.VMEM((n_chunks, CHUNK), jnp.int32)]
            + [pltpu.VMEM((CHUNK, D_MODEL), jnp.float32) for _ in range(NBUF)]
            + [pltpu.VMEM((2, D_MODEL), jnp.float32)]
            + [pltpu.SemaphoreType.DMA for _ in range(2 * NBUF)]
        ),
    )(_body)
    out = kern(x2d, table, gamma, beta)
    return out.reshape(b, s, D_MODEL)


# speedup vs baseline: 1.0516x; 1.0018x over previous
import functools

import jax
import jax.numpy as jnp
from jax import lax
from jax.experimental import pallas as pl
from jax.experimental.pallas import tpu as pltpu
from jax.experimental.pallas import tpu_sc as plsc

D_MODEL = 64
LANES = 16
CHUNK = 128
EPS_OVER_D = 1e-5 / 64.0
NBUF = 4
UNROLL = 8
NEWTON_ITERS = 2


def _body(x_hbm, table_hbm, gamma_hbm, beta_hbm, out_hbm,
          idx_v, rv0, rv1, rv2, rv3, gb_v,
          sg0, sg1, sg2, sg3, ss0, ss1, ss2, ss3):
    nc = 2
    wid = lax.axis_index("s") * nc + lax.axis_index("c")
    n_chunks = idx_v.shape[0]
    base = wid * n_chunks
    rvs = [rv0, rv1, rv2, rv3]
    sgs = [sg0, sg1, sg2, sg3]
    sss = [ss0, ss1, ss2, ss3]

    pltpu.sync_copy(x_hbm.at[pl.ds(base, n_chunks)], idx_v)
    pltpu.sync_copy(gamma_hbm, gb_v.at[0])
    pltpu.sync_copy(beta_hbm, gb_v.at[1])

    nq = D_MODEL // LANES
    last_lane = jnp.full((LANES,), LANES - 1, jnp.int32)
    gvecs = [gb_v[0, pl.ds(k * LANES, LANES)] for k in range(nq)]
    bvecs = [gb_v[1, pl.ds(k * LANES, LANES)] for k in range(nq)]

    def g_copy(j, p):
        return pltpu.make_async_copy(table_hbm.at[idx_v.at[j]], rvs[p], sgs[p])

    def s_copy(j, p):
        return pltpu.make_async_copy(
            rvs[p], out_hbm.at[pl.ds((base + j) * CHUNK, CHUNK)], sss[p])

    def compute(rows_v):
        def row_block(i, c):
            for u in range(UNROLL):
                r = i * UNROLL + u
                qs = [rows_v[r, pl.ds(k * LANES, LANES)] for k in range(nq)]
                t = (qs[0] + qs[1]) + (qs[2] + qs[3])
                t2 = (qs[0] * qs[0] + qs[1] * qs[1]) + (
                    qs[2] * qs[2] + qs[3] * qs[3])
                s = jnp.take_along_axis(jnp.cumsum(t), last_lane, axis=0)
                s2 = jnp.take_along_axis(jnp.cumsum(t2), last_lane, axis=0)
                mean = s * (1.0 / D_MODEL)
                var = s2 * (1.0 / D_MODEL) - mean * mean
                tv = var + EPS_OVER_D
                ti = plsc.bitcast(tv, jnp.int32)
                yi = 0x5F3759DF - lax.shift_right_logical(ti, 1)
                y = plsc.bitcast(yi, jnp.float32)
                half_t = tv * 0.5
                for _ in range(NEWTON_ITERS):
                    y = y * (1.5 - half_t * y * y)
                cshift = mean * y
                for k in range(nq):
                    o = qs[k] * y - cshift
                    rows_v[r, pl.ds(k * LANES, LANES)] = o * gvecs[k] + bvecs[k]
            return c

        lax.fori_loop(0, CHUNK // UNROLL, row_block, 0)

    g_copy(0, 0).start()
    g_copy(1, 1).start()

    n_outer = n_chunks // NBUF

    def outer(jj, carry):
        for p in range(NBUF):
            j = jj * NBUF + p
            q = (p + 2) % NBUF
            g_copy(j, p).wait()
            compute(rvs[p])
            s_copy(j, p).start()
            if p < 2:
                @pl.when(jj >= 1)
                def _wait_prev():
                    s_copy(j - 2, q).wait()
                g_copy(j + 2, q).start()
            else:
                s_copy(j - 2, q).wait()

                @pl.when(jj <= n_outer - 2)
                def _start_next():
                    g_copy(j + 2, q).start()
        return carry

    lax.fori_loop(0, n_outer, outer, 0)
    s_copy(n_chunks - 2, 2).wait()
    s_copy(n_chunks - 1, 3).wait()


def kernel(x, table, gamma, beta):
    b, s = x.shape
    n_tok = b * s
    n_workers = 32
    per_worker = n_tok // n_workers
    n_chunks = per_worker // CHUNK
    x2d = x.reshape(n_tok // CHUNK, CHUNK).astype(jnp.int32)

    mesh = plsc.VectorSubcoreMesh(core_axis_name="c", subcore_axis_name="s")
    kern = functools.partial(
        pl.kernel,
        mesh=mesh,
        compiler_params=pltpu.CompilerParams(
            use_tc_tiling_on_sc=False, needs_layout_passes=False),
        out_type=jax.ShapeDtypeStruct((n_tok, D_MODEL), jnp.float32),
        scratch_types=(
            [pltpu.VMEM((n_chunks, CHUNK), jnp.int32)]
            + [pltpu.VMEM((CHUNK, D_MODEL), jnp.float32) for _ in range(NBUF)]
            + [pltpu.VMEM((2, D_MODEL), jnp.float32)]
            + [pltpu.SemaphoreType.DMA for _ in range(2 * NBUF)]
        ),
    )(_body)
    out = kern(x2d, table, gamma, beta)
    return out.reshape(b, s, D_MODEL)


# speedup vs baseline: 1.0888x; 1.0353x over previous
import functools

import jax
import jax.numpy as jnp
from jax import lax
from jax.experimental import pallas as pl
from jax.experimental.pallas import tpu as pltpu
from jax.experimental.pallas import tpu_sc as plsc

D_MODEL = 64
LANES = 16
CHUNK = 128
EPS_OVER_D = 1e-5 / 64.0
NBUF = 4
UNROLL = 8
NEWTON_ITERS = 2


def _body(x_hbm, table_hbm, gamma_hbm, beta_hbm, out_hbm,
          idx_v, rv0, rv1, rv2, rv3, oi0, oi1, oi2, oi3, gb_v,
          sg0, sg1, sg2, sg3, ss0, ss1, ss2, ss3):
    nc = 2
    wid = lax.axis_index("s") * nc + lax.axis_index("c")
    n_chunks = idx_v.shape[0]
    base = wid * n_chunks
    rvs = [rv0, rv1, rv2, rv3]
    ois = [oi0, oi1, oi2, oi3]
    sgs = [sg0, sg1, sg2, sg3]
    sss = [ss0, ss1, ss2, ss3]
    seq_len = 50
    n_batch = 16384

    pltpu.sync_copy(x_hbm.at[pl.ds(base, n_chunks)], idx_v)
    pltpu.sync_copy(gamma_hbm, gb_v.at[0])
    pltpu.sync_copy(beta_hbm, gb_v.at[1])

    nq = D_MODEL // LANES
    last_lane = jnp.full((LANES,), LANES - 1, jnp.int32)
    gvecs = [gb_v[0, pl.ds(k * LANES, LANES)] for k in range(nq)]
    bvecs = [gb_v[1, pl.ds(k * LANES, LANES)] for k in range(nq)]

    def g_copy(j, p):
        return pltpu.make_async_copy(table_hbm.at[idx_v.at[j]], rvs[p], sgs[p])

    def s_copy(j, p):
        return pltpu.make_async_copy(
            rvs[p], out_hbm.at[ois[p].at[0]], sss[p])

    iota16 = lax.iota(jnp.int32, LANES)

    def fill_out_idx(j, p):
        t0 = (base + j) * CHUNK
        for g in range(CHUNK // LANES):
            t = t0 + g * LANES + iota16
            ois[p][0, pl.ds(g * LANES, LANES)] = (
                lax.rem(t, seq_len) * n_batch + lax.div(t, seq_len))

    def compute(rows_v):
        def row_block(i, c):
            for u in range(UNROLL):
                r = i * UNROLL + u
                qs = [rows_v[r, pl.ds(k * LANES, LANES)] for k in range(nq)]
                t = (qs[0] + qs[1]) + (qs[2] + qs[3])
                t2 = (qs[0] * qs[0] + qs[1] * qs[1]) + (
                    qs[2] * qs[2] + qs[3] * qs[3])
                s = jnp.take_along_axis(jnp.cumsum(t), last_lane, axis=0)
                s2 = jnp.take_along_axis(jnp.cumsum(t2), last_lane, axis=0)
                mean = s * (1.0 / D_MODEL)
                var = s2 * (1.0 / D_MODEL) - mean * mean
                tv = var + EPS_OVER_D
                ti = plsc.bitcast(tv, jnp.int32)
                yi = 0x5F3759DF - lax.shift_right_logical(ti, 1)
                y = plsc.bitcast(yi, jnp.float32)
                half_t = tv * 0.5
                for _ in range(NEWTON_ITERS):
                    y = y * (1.5 - half_t * y * y)
                cshift = mean * y
                for k in range(nq):
                    o = qs[k] * y - cshift
                    rows_v[r, pl.ds(k * LANES, LANES)] = o * gvecs[k] + bvecs[k]
            return c

        lax.fori_loop(0, CHUNK // UNROLL, row_block, 0)

    g_copy(0, 0).start()
    g_copy(1, 1).start()

    n_outer = n_chunks // NBUF

    def outer(jj, carry):
        for p in range(NBUF):
            j = jj * NBUF + p
            q = (p + 2) % NBUF
            g_copy(j, p).wait()
            compute(rvs[p])
            fill_out_idx(j, p)
            s_copy(j, p).start()
            if p < 2:
                @pl.when(jj >= 1)
                def _wait_prev():
                    s_copy(j - 2, q).wait()
                g_copy(j + 2, q).start()
            else:
                s_copy(j - 2, q).wait()

                @pl.when(jj <= n_outer - 2)
                def _start_next():
                    g_copy(j + 2, q).start()
        return carry

    lax.fori_loop(0, n_outer, outer, 0)
    s_copy(n_chunks - 2, 2).wait()
    s_copy(n_chunks - 1, 3).wait()


def kernel(x, table, gamma, beta):
    b, s = x.shape
    n_tok = b * s
    n_workers = 32
    per_worker = n_tok // n_workers
    n_chunks = per_worker // CHUNK
    x2d = x.reshape(n_tok // CHUNK, CHUNK).astype(jnp.int32)

    mesh = plsc.VectorSubcoreMesh(core_axis_name="c", subcore_axis_name="s")
    kern = functools.partial(
        pl.kernel,
        mesh=mesh,
        compiler_params=pltpu.CompilerParams(
            use_tc_tiling_on_sc=False, needs_layout_passes=False),
        out_type=jax.ShapeDtypeStruct((n_tok, D_MODEL), jnp.float32),
        scratch_types=(
            [pltpu.VMEM((n_chunks, CHUNK), jnp.int32)]
            + [pltpu.VMEM((CHUNK, D_MODEL), jnp.float32) for _ in range(NBUF)]
            + [pltpu.VMEM((1, CHUNK), jnp.int32) for _ in range(NBUF)]
            + [pltpu.VMEM((2, D_MODEL), jnp.float32)]
            + [pltpu.SemaphoreType.DMA for _ in range(2 * NBUF)]
        ),
    )(_body)
    out = kern(x2d, table, gamma, beta)
    return out.reshape(s, b, D_MODEL).transpose(1, 0, 2)


# speedup vs baseline: 1.0890x; 1.0002x over previous
import functools

import jax
import jax.numpy as jnp
from jax import lax
from jax.experimental import pallas as pl
from jax.experimental.pallas import tpu as pltpu
from jax.experimental.pallas import tpu_sc as plsc

D_MODEL = 64
LANES = 16
SEQ_LEN = 50
N_BATCH = 16384
CHUNK = 128
EPS_OVER_D = 1e-5 / 64.0
NBUF = 4
UNROLL = 8
NEWTON_ITERS = 2


def _body(x_hbm, table_hbm, gamma_hbm, beta_hbm, out_hbm,
          idx_v, rv0, rv1, rv2, rv3, oi0, oi1, oi2, oi3, gb_v,
          sg0, sg1, sg2, sg3, ss0, ss1, ss2, ss3):
    nc = 2
    wid = lax.axis_index("s") * nc + lax.axis_index("c")
    n_chunks = idx_v.shape[0]
    base = wid * n_chunks
    rvs = [rv0, rv1, rv2, rv3]
    ois = [oi0, oi1, oi2, oi3]
    sgs = [sg0, sg1, sg2, sg3]
    sss = [ss0, ss1, ss2, ss3]

    pltpu.sync_copy(x_hbm.at[pl.ds(base, n_chunks)], idx_v)
    pltpu.sync_copy(gamma_hbm, gb_v.at[0])
    pltpu.sync_copy(beta_hbm, gb_v.at[1])

    nq = D_MODEL // LANES
    last_lane = jnp.full((LANES,), LANES - 1, jnp.int32)
    gvecs = [gb_v[0, pl.ds(k * LANES, LANES)] for k in range(nq)]
    bvecs = [gb_v[1, pl.ds(k * LANES, LANES)] for k in range(nq)]

    def g_copy(j, p):
        return pltpu.make_async_copy(table_hbm.at[idx_v.at[j]], rvs[p], sgs[p])

    def s_copy(j, p):
        return pltpu.make_async_copy(
            rvs[p], out_hbm.at[ois[p].at[0]], sss[p])

    iota16 = lax.iota(jnp.int32, LANES)

    def fill_out_idx(j, p):
        t0 = (base + j) * CHUNK
        for g in range(CHUNK // LANES):
            t = t0 + g * LANES + iota16
            ois[p][0, pl.ds(g * LANES, LANES)] = (
                lax.rem(t, SEQ_LEN) * N_BATCH + lax.div(t, SEQ_LEN))

    def compute(rows_v):
        def row_block(i, c):
            for u in range(UNROLL):
                r = i * UNROLL + u
                qs = [rows_v[r, pl.ds(k * LANES, LANES)] for k in range(nq)]
                t = (qs[0] + qs[1]) + (qs[2] + qs[3])
                t2 = (qs[0] * qs[0] + qs[1] * qs[1]) + (
                    qs[2] * qs[2] + qs[3] * qs[3])
                s = jnp.take_along_axis(jnp.cumsum(t), last_lane, axis=0)
                s2 = jnp.take_along_axis(jnp.cumsum(t2), last_lane, axis=0)
                mean = s * (1.0 / D_MODEL)
                var = s2 * (1.0 / D_MODEL) - mean * mean
                tv = var + EPS_OVER_D
                ti = plsc.bitcast(tv, jnp.int32)
                yi = 0x5F3759DF - lax.shift_right_logical(ti, 1)
                y = plsc.bitcast(yi, jnp.float32)
                half_t = tv * 0.5
                for _ in range(NEWTON_ITERS):
                    y = y * (1.5 - half_t * y * y)
                cshift = mean * y
                for k in range(nq):
                    o = qs[k] * y - cshift
                    rows_v[r, pl.ds(k * LANES, LANES)] = o * gvecs[k] + bvecs[k]
            return c

        lax.fori_loop(0, CHUNK // UNROLL, row_block, 0)

    g_copy(0, 0).start()
    g_copy(1, 1).start()

    n_outer = n_chunks // NBUF

    def outer(jj, carry):
        for p in range(NBUF):
            j = jj * NBUF + p
            q = (p + 2) % NBUF
            g_copy(j, p).wait()
            compute(rvs[p])
            fill_out_idx(j, p)
            s_copy(j, p).start()
            if p < 2:
                @pl.when(jj >= 1)
                def _wait_prev():
                    s_copy(j - 2, q).wait()
                g_copy(j + 2, q).start()
            else:
                s_copy(j - 2, q).wait()

                @pl.when(jj <= n_outer - 2)
                def _start_next():
                    g_copy(j + 2, q).start()
        return carry

    lax.fori_loop(0, n_outer, outer, 0)
    s_copy(n_chunks - 2, 2).wait()
    s_copy(n_chunks - 1, 3).wait()


def kernel(x, table, gamma, beta):
    b, s = x.shape
    n_tok = b * s
    n_workers = 32
    per_worker = n_tok // n_workers
    n_chunks = per_worker // CHUNK
    x2d = x.reshape(n_tok // CHUNK, CHUNK).astype(jnp.int32)

    mesh = plsc.VectorSubcoreMesh(core_axis_name="c", subcore_axis_name="s")
    kern = functools.partial(
        pl.kernel,
        mesh=mesh,
        compiler_params=pltpu.CompilerParams(
            use_tc_tiling_on_sc=False, needs_layout_passes=False),
        out_type=jax.ShapeDtypeStruct((n_tok, D_MODEL), jnp.float32),
        scratch_types=(
            [pltpu.VMEM((n_chunks, CHUNK), jnp.int32)]
            + [pltpu.VMEM((CHUNK, D_MODEL), jnp.float32) for _ in range(NBUF)]
            + [pltpu.VMEM((1, CHUNK), jnp.int32) for _ in range(NBUF)]
            + [pltpu.VMEM((2, D_MODEL), jnp.float32)]
            + [pltpu.SemaphoreType.DMA for _ in range(2 * NBUF)]
        ),
    )(_body)
    out = kern(x2d, table, gamma, beta)
    return out.reshape(s, b, D_MODEL).transpose(1, 0, 2)
